# baseline (speedup 1.0000x reference)
import jax
import jax.numpy as jnp
from jax.experimental import pallas as pl
from jax.experimental.pallas import tpu as pltpu

_SIZE = 65536
_DIM = 128
_BATCH = 1024
_WIDTHS = (16384, 49152)
_STARTS = tuple(sum(_WIDTHS[:j]) for j in range(len(_WIDTHS)))
_NCH = len(_WIDTHS)


def _body(out_hbm, bank_hbm, oo_hbm, snap_hbm, new_hbm, *scratch):
    bufs = scratch[:_NCH]
    vin, vout, isem, ssem, nsem, osem = scratch[_NCH:]

    def in_cp(j):
        c, w = _STARTS[j], _WIDTHS[j]
        return pltpu.make_async_copy(
            bank_hbm.at[:, pl.ds(c, w)], bufs[j], isem.at[j])

    def snap_cp(j):
        c, w = _STARTS[j], _WIDTHS[j]
        return pltpu.make_async_copy(
            bufs[j], snap_hbm.at[:, pl.ds(c, w)], ssem.at[j])

    def new_cp(j):
        if j == 0:
            return pltpu.make_async_copy(
                bufs[0].at[:, pl.ds(_BATCH, _WIDTHS[0] - _BATCH)],
                new_hbm.at[:, pl.ds(_BATCH, _WIDTHS[0] - _BATCH)], nsem.at[0])
        c, w = _STARTS[j], _WIDTHS[j]
        return pltpu.make_async_copy(
            bufs[j], new_hbm.at[:, pl.ds(c, w)], nsem.at[j])

    ocp_in = pltpu.make_async_copy(out_hbm, vin, osem.at[0])
    ocp_in.start()
    for j in range(_NCH):
        in_cp(j).start()
    ocp_in.wait()
    vout[...] = jnp.transpose(vin[...])
    pltpu.make_async_copy(vin, oo_hbm, osem.at[1]).start()
    pltpu.make_async_copy(vout, new_hbm.at[:, pl.ds(0, _BATCH)], osem.at[2]).start()
    for j in range(_NCH):
        in_cp(j).wait()
        snap_cp(j).start()
        new_cp(j).start()
    for j in range(_NCH):
        snap_cp(j).wait()
        new_cp(j).wait()
    pltpu.make_async_copy(vin, oo_hbm, osem.at[1]).wait()
    pltpu.make_async_copy(vout, new_hbm.at[:, pl.ds(0, _BATCH)], osem.at[2]).wait()


def kernel(output, bank):
    out_shapes = (
        jax.ShapeDtypeStruct((_BATCH, _DIM), output.dtype),
        jax.ShapeDtypeStruct((_DIM, _SIZE), bank.dtype),
        jax.ShapeDtypeStruct((_DIM, _SIZE), bank.dtype),
    )
    out, snap, new = pl.pallas_call(
        _body,
        in_specs=[
            pl.BlockSpec(memory_space=pl.ANY),
            pl.BlockSpec(memory_space=pl.ANY),
        ],
        out_specs=[
            pl.BlockSpec(memory_space=pl.ANY),
            pl.BlockSpec(memory_space=pl.ANY),
            pl.BlockSpec(memory_space=pl.ANY),
        ],
        out_shape=out_shapes,
        scratch_shapes=(
            [pltpu.VMEM((_DIM, w), jnp.float32) for w in _WIDTHS]
            + [
                pltpu.VMEM((_BATCH, _DIM), jnp.float32),
                pltpu.VMEM((_DIM, _BATCH), jnp.float32),
                pltpu.SemaphoreType.DMA((_NCH,)),
                pltpu.SemaphoreType.DMA((_NCH,)),
                pltpu.SemaphoreType.DMA((_NCH,)),
                pltpu.SemaphoreType.DMA((3,)),
            ]
        ),
    )(output, bank)
    return (out, snap, new)


# speedup vs baseline: 1.1304x; 1.1304x over previous
import jax
import jax.numpy as jnp
from jax.experimental import pallas as pl
from jax.experimental.pallas import tpu as pltpu

_SIZE = 65536
_DIM = 128
_BATCH = 1024
_WIDTHS = (32768, 32768)
_STARTS = tuple(sum(_WIDTHS[:j]) for j in range(len(_WIDTHS)))
_NCH = len(_WIDTHS)


def _body(out_hbm, bank_hbm, oo_hbm, snap_hbm, new_hbm, *scratch):
    bufs = scratch[:_NCH]
    vin, vout, isem, ssem, nsem, osem = scratch[_NCH:]

    def in_cp(j):
        c, w = _STARTS[j], _WIDTHS[j]
        return pltpu.make_async_copy(
            bank_hbm.at[:, pl.ds(c, w)], bufs[j], isem.at[j])

    def snap_cp(j):
        c, w = _STARTS[j], _WIDTHS[j]
        return pltpu.make_async_copy(
            bufs[j], snap_hbm.at[:, pl.ds(c, w)], ssem.at[j])

    def new_cp(j):
        if j == 0:
            return pltpu.make_async_copy(
                bufs[0].at[:, pl.ds(_BATCH, _WIDTHS[0] - _BATCH)],
                new_hbm.at[:, pl.ds(_BATCH, _WIDTHS[0] - _BATCH)], nsem.at[0])
        c, w = _STARTS[j], _WIDTHS[j]
        return pltpu.make_async_copy(
            bufs[j], new_hbm.at[:, pl.ds(c, w)], nsem.at[j])

    ocp_in = pltpu.make_async_copy(out_hbm, vin, osem.at[0])
    ocp_in.start()
    for j in range(_NCH):
        in_cp(j).start()
    ocp_in.wait()
    vout[...] = jnp.transpose(vin[...])
    pltpu.make_async_copy(vin, oo_hbm, osem.at[1]).start()
    pltpu.make_async_copy(vout, new_hbm.at[:, pl.ds(0, _BATCH)], osem.at[2]).start()
    for j in range(_NCH):
        in_cp(j).wait()
        snap_cp(j).start()
        new_cp(j).start()
    for j in range(_NCH):
        snap_cp(j).wait()
        new_cp(j).wait()
    pltpu.make_async_copy(vin, oo_hbm, osem.at[1]).wait()
    pltpu.make_async_copy(vout, new_hbm.at[:, pl.ds(0, _BATCH)], osem.at[2]).wait()


def kernel(output, bank):
    out_shapes = (
        jax.ShapeDtypeStruct((_BATCH, _DIM), output.dtype),
        jax.ShapeDtypeStruct((_DIM, _SIZE), bank.dtype),
        jax.ShapeDtypeStruct((_DIM, _SIZE), bank.dtype),
    )
    out, snap, new = pl.pallas_call(
        _body,
        in_specs=[
            pl.BlockSpec(memory_space=pl.ANY),
            pl.BlockSpec(memory_space=pl.ANY),
        ],
        out_specs=[
            pl.BlockSpec(memory_space=pl.ANY),
            pl.BlockSpec(memory_space=pl.ANY),
            pl.BlockSpec(memory_space=pl.ANY),
        ],
        out_shape=out_shapes,
        scratch_shapes=(
            [pltpu.VMEM((_DIM, w), jnp.float32) for w in _WIDTHS]
            + [
                pltpu.VMEM((_BATCH, _DIM), jnp.float32),
                pltpu.VMEM((_DIM, _BATCH), jnp.float32),
                pltpu.SemaphoreType.DMA((_NCH,)),
                pltpu.SemaphoreType.DMA((_NCH,)),
                pltpu.SemaphoreType.DMA((_NCH,)),
                pltpu.SemaphoreType.DMA((3,)),
            ]
        ),
    )(output, bank)
    return (out, snap, new)
